# Initial kernel scaffold; baseline (speedup 1.0000x reference)
import jax
import jax.numpy as jnp
from jax.experimental import pallas as pl


def kernel(vocab_ids, table):
    raise NotImplementedError("write your pallas kernel here")


# speedup vs baseline: 3.5731x; 3.5731x over previous
import functools

import jax
import jax.numpy as jnp
from jax import lax
from jax.experimental import pallas as pl
from jax.experimental.pallas import tpu as pltpu
from jax.experimental.pallas import tpu_sc as plsc

_VOCAB = 1000
_DIM = 64
_BATCH = 4096
_HIST = 200
_N = _BATCH * _HIST

_NC = 2
_NS = 16
_NW = _NC * _NS

_ROWS_PER_W = _N // _NW
_CHUNK = 512
_IDX_W = 128
_IPC = _CHUNK // _IDX_W
_NCHUNK = _ROWS_PER_W // _CHUNK
_IDXROWS_PER_W = _ROWS_PER_W // _IDX_W

_mesh = plsc.VectorSubcoreMesh(core_axis_name="c", subcore_axis_name="s")


@functools.partial(
    pl.kernel,
    out_type=jax.ShapeDtypeStruct((_N, _DIM), jnp.float32),
    mesh=_mesh,
    scratch_types=[
        pltpu.VMEM((_IPC, _IDX_W), jnp.int32),
        pltpu.VMEM((_CHUNK, _DIM), jnp.float32),
        pltpu.SemaphoreType.DMA,
    ],
    compiler_params=pltpu.CompilerParams(use_tc_tiling_on_sc=False),
)
def _emb_lookup(idx_hbm, table_hbm, out_hbm, idx_v, rows_v, sem):
    wid = lax.axis_index("s") * _NC + lax.axis_index("c")

    def chunk_body(i, carry):
        r0 = wid * _ROWS_PER_W + i * _CHUNK
        ir0 = wid * _IDXROWS_PER_W + i * _IPC
        pltpu.sync_copy(idx_hbm.at[pl.ds(ir0, _IPC)], idx_v)
        copies = [
            pltpu.async_copy(
                table_hbm.at[idx_v.at[j]],
                rows_v.at[pl.ds(j * _IDX_W, _IDX_W)],
                sem,
            )
            for j in range(_IPC)
        ]
        for c in copies:
            c.wait()
        pltpu.sync_copy(rows_v, out_hbm.at[pl.ds(r0, _CHUNK)])
        return carry

    lax.fori_loop(0, _NCHUNK, chunk_body, 0)


def kernel(vocab_ids, table):
    idx = vocab_ids.reshape(_N).astype(jnp.int32)
    idx2d = idx.reshape(_N // _IDX_W, _IDX_W)
    out = _emb_lookup(idx2d, table)
    return out.reshape(_BATCH, _HIST, _DIM)


# speedup vs baseline: 3.5926x; 1.0055x over previous
import functools

import jax
import jax.numpy as jnp
from jax import lax
from jax.experimental import pallas as pl
from jax.experimental.pallas import tpu as pltpu
from jax.experimental.pallas import tpu_sc as plsc

_VOCAB = 1000
_DIM = 64
_BATCH = 4096
_HIST = 200
_N = _BATCH * _HIST

_NC = 2
_NS = 16
_NW = _NC * _NS

_ROWS_PER_W = _N // _NW
_CHUNK = 512
_IDX_W = 128
_IPC = _CHUNK // _IDX_W
_NCHUNK = _ROWS_PER_W // _CHUNK
_NT = _NCHUNK // 2
_IDXROWS_PER_W = _ROWS_PER_W // _IDX_W

_mesh = plsc.VectorSubcoreMesh(core_axis_name="c", subcore_axis_name="s")


@functools.partial(
    pl.kernel,
    out_type=jax.ShapeDtypeStruct((_N, _DIM), jnp.float32),
    mesh=_mesh,
    scratch_types=[
        pltpu.VMEM((_IPC, _IDX_W), jnp.int32),
        pltpu.VMEM((_IPC, _IDX_W), jnp.int32),
        pltpu.VMEM((_CHUNK, _DIM), jnp.float32),
        pltpu.VMEM((_CHUNK, _DIM), jnp.float32),
        pltpu.SemaphoreType.DMA,
        pltpu.SemaphoreType.DMA,
        pltpu.SemaphoreType.DMA,
    ],
    compiler_params=pltpu.CompilerParams(use_tc_tiling_on_sc=False),
)
def _emb_lookup(idx_hbm, table_hbm, out_hbm,
                idx0, idx1, rows0, rows1, sem_g, sem_w0, sem_w1):
    wid = lax.axis_index("s") * _NC + lax.axis_index("c")
    row_base = wid * _ROWS_PER_W
    irow_base = wid * _IDXROWS_PER_W

    def load_idx(i, idx_v):
        pltpu.sync_copy(idx_hbm.at[pl.ds(irow_base + i * _IPC, _IPC)], idx_v)

    def fire_gathers(idx_v, rows_v):
        for j in range(_IPC):
            pltpu.async_copy(table_hbm.at[idx_v.at[j]],
                             rows_v.at[pl.ds(j * _IDX_W, _IDX_W)], sem_g)

    def drain_gathers(idx_v, rows_v):
        for j in range(_IPC):
            pltpu.make_async_copy(table_hbm.at[idx_v.at[j]],
                                  rows_v.at[pl.ds(j * _IDX_W, _IDX_W)],
                                  sem_g).wait()

    def fire_writeout(i, rows_v, sem):
        pltpu.async_copy(rows_v, out_hbm.at[pl.ds(row_base + i * _CHUNK, _CHUNK)], sem)

    def wait_writeout(i, rows_v, sem):
        pltpu.make_async_copy(rows_v, out_hbm.at[pl.ds(row_base + i * _CHUNK, _CHUNK)],
                              sem).wait()

    load_idx(0, idx0)
    fire_gathers(idx0, rows0)

    def body(t, carry):
        a = 2 * t
        b = a + 1
        load_idx(b, idx1)
        drain_gathers(idx0, rows0)

        @pl.when(t > 0)
        def _():
            wait_writeout(b - 2, rows1, sem_w1)

        fire_gathers(idx1, rows1)
        fire_writeout(a, rows0, sem_w0)

        @pl.when(t < _NT - 1)
        def _():
            load_idx(a + 2, idx0)

        drain_gathers(idx1, rows1)
        fire_writeout(b, rows1, sem_w1)
        wait_writeout(a, rows0, sem_w0)

        @pl.when(t < _NT - 1)
        def _():
            fire_gathers(idx0, rows0)

        return carry

    lax.fori_loop(0, _NT, body, 0)
    wait_writeout(_NCHUNK - 1, rows1, sem_w1)


def kernel(vocab_ids, table):
    idx = vocab_ids.reshape(_N).astype(jnp.int32)
    idx2d = idx.reshape(_N // _IDX_W, _IDX_W)
    out = _emb_lookup(idx2d, table)
    return out.reshape(_BATCH, _HIST, _DIM)
